# baseline (speedup 1.0000x reference)
import functools

import jax
import jax.numpy as jnp
from jax import lax
from jax.experimental import pallas as pl
from jax.experimental.pallas import tpu as pltpu
from jax.experimental.pallas import tpu_sc as plsc

E = 8
TOPK = 2
D_MODEL = 2048
EXP_DIM = 4096
N_TOK = 8192
K_CAP = (N_TOK * TOPK) // E
TILE = 512
N_TILES = N_TOK // TILE


def _router_body(x_ref, w_ref, lt_ref, wt_ref):
    lt = jax.lax.dot_general(
        w_ref[...], x_ref[...].astype(jnp.bfloat16), (((1,), (1,)), ((), ())),
        preferred_element_type=jnp.float32)
    lt_ref[...] = lt
    m = jnp.max(lt, axis=0, keepdims=True)
    ex = jnp.exp(lt - m)
    wt_ref[...] = ex / jnp.sum(ex, axis=0, keepdims=True)


def _router(x_flat, W_router):
    return pl.pallas_call(
        _router_body,
        grid=(N_TILES,),
        in_specs=[pl.BlockSpec((TILE, D_MODEL), lambda i: (i, 0)),
                  pl.BlockSpec((E, D_MODEL), lambda i: (0, 0))],
        out_specs=[pl.BlockSpec((E, TILE), lambda i: (0, i)),
                   pl.BlockSpec((E, TILE), lambda i: (0, i))],
        out_shape=[jax.ShapeDtypeStruct((E, N_TOK), jnp.float32),
                   jax.ShapeDtypeStruct((E, N_TOK), jnp.float32)],
    )(x_flat, W_router)


def _rank_body(lcol_ref, lrow_ref, rt_ref):
    li = lrow_ref[0]
    i0 = pl.program_id(1) * TILE
    ivec = i0 + jax.lax.broadcasted_iota(jnp.int32, (TILE, TILE), 1)
    jiota = jax.lax.broadcasted_iota(jnp.int32, (TILE, TILE), 0)
    cnt = jnp.zeros((1, TILE), jnp.int32)
    for jb in range(N_TILES):
        lj = lcol_ref[0, jb * TILE:(jb + 1) * TILE, :]
        gt = lj > li
        tie = (lj == li) & ((jb * TILE + jiota) < ivec)
        cnt += jnp.sum((gt | tie).astype(jnp.int32), axis=0, keepdims=True)
    rt_ref[0] = cnt


def _rank(logits3, logits_rows, n_tok):
    rt3 = pl.pallas_call(
        _rank_body,
        grid=(E, N_TILES),
        in_specs=[pl.BlockSpec((1, n_tok, 1), lambda e, i: (e, 0, 0)),
                  pl.BlockSpec((1, 1, TILE), lambda e, i: (e * N_TILES + i, 0, 0))],
        out_specs=pl.BlockSpec((1, 1, TILE), lambda e, i: (e * N_TILES + i, 0, 0)),
        out_shape=jax.ShapeDtypeStruct((E * N_TILES, 1, TILE), jnp.int32),
    )(logits3, logits_rows)
    return rt3.reshape(E, n_tok)


def _fan_body(rt_ref, fan_ref):
    fan_ref[...] = jnp.sum((rt_ref[...] < K_CAP).astype(jnp.float32),
                           axis=0, keepdims=True)


def _fanout(rt):
    return pl.pallas_call(
        _fan_body,
        grid=(N_TILES,),
        in_specs=[pl.BlockSpec((E, TILE), lambda i: (0, i))],
        out_specs=pl.BlockSpec((1, TILE), lambda i: (0, i)),
        out_shape=jax.ShapeDtypeStruct((1, N_TOK), jnp.float32),
    )(rt)


_SC_CHUNK = 16
_SLICE = 512
_TQ = N_TOK // 4
_PAD = 4096


def _make_sc_kernel():
    info = plsc.get_sparse_core_info()
    nc = info.num_cores
    mesh = plsc.VectorSubcoreMesh(core_axis_name="c", subcore_axis_name="s")
    flat = E * K_CAP

    nch = _SLICE // _SC_CHUNK
    _REG = 1024

    @functools.partial(
        pl.kernel, mesh=mesh,
        out_type=[jax.ShapeDtypeStruct((flat // 16, 16), jnp.int32),
                  jax.ShapeDtypeStruct((flat // 16, 16), jnp.float32),
                  jax.ShapeDtypeStruct((flat // 16, 16), jnp.float32),
                  jax.ShapeDtypeStruct((flat, D_MODEL), jnp.float32)],
        scratch_types=[pltpu.VMEM((N_TOK,), jnp.int32),
                       pltpu.VMEM((N_TOK,), jnp.float32),
                       pltpu.VMEM((N_TOK,), jnp.float32),
                       pltpu.VMEM((N_TOK // 128, 128), jnp.int32),
                       pltpu.VMEM((N_TOK,), jnp.int32),
                       pltpu.VMEM((nch, _SC_CHUNK), jnp.int32),
                       pltpu.VMEM((nch, _SC_CHUNK), jnp.float32),
                       pltpu.VMEM((nch, _SC_CHUNK), jnp.float32),
                       pltpu.VMEM((_SC_CHUNK, D_MODEL), jnp.float32),
                       pltpu.VMEM_SHARED((16 * _REG,), jnp.int32),
                       pltpu.VMEM_SHARED((16 * _REG,), jnp.float32),
                       pltpu.VMEM_SHARED((16 * _REG,), jnp.float32),
                       pltpu.SemaphoreType.DMA],
    )
    def sc_kernel(rt_hbm, wt_hbm, lt_hbm, x_hbm,
                  idx_out, w_out, v_out, xsel_out,
                  rv, wv, lv, tgt2, tokv, idxb, wb, vb, rowbuf,
                  spm_i, spm_w, spm_v, sem):
        c = lax.axis_index("c")
        s = lax.axis_index("s")
        e = c * (E // nc) + s // 4
        q = s % 4
        rank_lo = q * _SLICE
        base = e * K_CAP
        row0 = pl.multiple_of(base + rank_lo, _SLICE)

        pltpu.sync_copy(rt_hbm.at[e], rv)
        pltpu.sync_copy(wt_hbm.at[e], wv)
        pltpu.sync_copy(lt_hbm.at[e], lv)

        iota16 = jax.lax.iota(jnp.int32, 16)
        reg0 = s * _REG

        def body(nb, _):
            r = rv[pl.ds(nb * 16, 16)]
            tok = nb * 16 + iota16
            in_q = (r >= rank_lo) & (r < rank_lo + _SLICE)
            loc = jnp.where(in_q, r - rank_lo, _SLICE + (tok & (_SLICE - 1)))
            tgt2[nb // 8, pl.ds((nb % 8) * 16, 16)] = reg0 + loc
            tokv[pl.ds(nb * 16, 16)] = tok
            return 0

        lax.fori_loop(0, N_TOK // 16, body, 0)

        for jj in range(N_TOK // 128):
            tg = tgt2.at[jj]
            sl = pl.ds(jj * 128, 128)
            pltpu.sync_copy(tokv.at[sl], spm_i.at[tg])
            pltpu.sync_copy(wv.at[sl], spm_w.at[tg])
            pltpu.sync_copy(lv.at[sl], spm_v.at[tg])

        for j in range(nch):
            sl = pl.ds(reg0 + j * _SC_CHUNK, _SC_CHUNK)
            pltpu.sync_copy(spm_i.at[sl], idxb.at[j])
            pltpu.sync_copy(spm_w.at[sl], wb.at[j])
            pltpu.sync_copy(spm_v.at[sl], vb.at[j])

        ro16 = pl.multiple_of(row0 // 16, nch)
        pltpu.sync_copy(idxb, idx_out.at[pl.ds(ro16, nch)])
        pltpu.sync_copy(wb, w_out.at[pl.ds(ro16, nch)])
        pltpu.sync_copy(vb, v_out.at[pl.ds(ro16, nch)])

        for j in range(nch):
            pltpu.async_copy(x_hbm.at[idxb.at[j]], rowbuf, sem).wait()
            pltpu.sync_copy(rowbuf,
                            xsel_out.at[pl.ds(row0 + j * _SC_CHUNK, _SC_CHUNK)])

    return sc_kernel


def _mlp1_body(x_ref, w1_ref, h_ref):
    xb = x_ref[...].astype(jnp.bfloat16)
    h1 = jax.lax.dot_general(
        xb, w1_ref[0], (((1,), (1,)), ((), ())),
        preferred_element_type=jnp.float32)
    h_ref[...] = jnp.square(jnp.maximum(h1, 0.0)).astype(jnp.bfloat16)


def _mlp2_body(h_ref, w2_ref, o_ref):
    o_ref[...] = jax.lax.dot_general(
        h_ref[...], w2_ref[0], (((1,), (1,)), ((), ())),
        preferred_element_type=jnp.float32)


def _mlp(x_sel, W1b, W2b):
    kt = K_CAP // TILE
    h1 = pl.pallas_call(
        _mlp1_body,
        grid=(E, kt),
        in_specs=[pl.BlockSpec((TILE, D_MODEL), lambda e, t: (e * 4 + t, 0)),
                  pl.BlockSpec((1, EXP_DIM, D_MODEL), lambda e, t: (e, 0, 0))],
        out_specs=pl.BlockSpec((TILE, EXP_DIM), lambda e, t: (e * 4 + t, 0)),
        out_shape=jax.ShapeDtypeStruct((E * K_CAP, EXP_DIM), jnp.bfloat16),
        compiler_params=pltpu.CompilerParams(
            dimension_semantics=("arbitrary", "arbitrary")),
    )(x_sel, W1b)
    return pl.pallas_call(
        _mlp2_body,
        grid=(E, kt),
        in_specs=[pl.BlockSpec((TILE, EXP_DIM), lambda e, t: (e * 4 + t, 0)),
                  pl.BlockSpec((1, D_MODEL, EXP_DIM), lambda e, t: (e, 0, 0))],
        out_specs=pl.BlockSpec((TILE, D_MODEL), lambda e, t: (e * 4 + t, 0)),
        out_shape=jax.ShapeDtypeStruct((E * K_CAP, D_MODEL), jnp.float32),
        compiler_params=pltpu.CompilerParams(
            dimension_semantics=("arbitrary", "arbitrary")),
    )(h1, W2b)


def kernel(x, W_router, W1, W2):
    bsz, seqlen, hidden = x.shape
    x_flat = x.reshape(-1, hidden)

    logitsT, weightsT = _router(x_flat, W_router)

    rt = _rank(logitsT.reshape(E, N_TOK, 1),
               logitsT.reshape(E * N_TILES, 1, TILE), N_TOK)
    fanout = _fanout(rt)

    sc = _make_sc_kernel()
    idx1, w1, v1, x_sel = sc(rt, weightsT, logitsT, x_flat)

    W1b = W1.astype(jnp.bfloat16)
    W2b = W2.astype(jnp.bfloat16)
    h_flat = _mlp(x_sel, W1b, W2b)

    indices_flat = idx1.reshape(-1)
    weights_flat = w1.reshape(-1)
    cutoffs = v1.reshape(-1)[K_CAP - 1::K_CAP]
    return h_flat, indices_flat, weights_flat, fanout.reshape(-1), cutoffs
